# Initial kernel scaffold; baseline (speedup 1.0000x reference)
import jax
import jax.numpy as jnp
from jax.experimental import pallas as pl


def kernel(premise, hypothesis, emb_table, W1, b1, W2, b2):
    raise NotImplementedError("write your pallas kernel here")


# speedup vs baseline: 5.5154x; 5.5154x over previous
import functools

import jax
import jax.numpy as jnp
from jax import lax
from jax.experimental import pallas as pl
from jax.experimental.pallas import tpu as pltpu
from jax.experimental.pallas import tpu_sc as plsc

VOCAB = 100000
D = 100
DP = 128
L = 20
B = 16384
NEX = 2 * B
HIDDEN = 4096

CEX = 32
ROWS_PER_CHUNK = CEX * L
GS = 128
NG = ROWS_PER_CHUNK // GS


def _make_sc_pool():
    info = plsc.get_sparse_core_info()
    nc, ns = info.num_cores, info.num_subcores
    nw = nc * ns
    ex_per_w = NEX // nw
    chunks_per_w = ex_per_w // CEX
    n_chunks = NEX // CEX

    mesh = plsc.VectorSubcoreMesh(core_axis_name="c", subcore_axis_name="s")

    @functools.partial(
        pl.kernel,
        mesh=mesh,
        out_type=jax.ShapeDtypeStruct((NEX, DP), jnp.float32),
        scratch_types=[
            pltpu.VMEM((ROWS_PER_CHUNK,), jnp.int32),
            pltpu.VMEM((ROWS_PER_CHUNK, DP), jnp.float32),
            pltpu.VMEM((CEX, DP), jnp.float32),
            pltpu.SemaphoreType.DMA,
        ],
    )
    def pool_kernel(table_hbm, idx_hbm, out_hbm, idx_v, rows_v, out_v, sem):
        wid = lax.axis_index("s") * nc + lax.axis_index("c")
        chunk0 = wid * chunks_per_w

        def chunk_body(c, carry):
            chunk = chunk0 + c
            ex0 = chunk * CEX
            pltpu.sync_copy(idx_hbm.at[chunk], idx_v)
            copies = []
            for g in range(NG):
                copies.append(
                    pltpu.async_copy(
                        table_hbm.at[idx_v.at[pl.ds(g * GS, GS)]],
                        rows_v.at[pl.ds(g * GS, GS)],
                        sem,
                    )
                )
            for cp in copies:
                cp.wait()

            def ex_body(e, carry2):
                r0 = e * L
                for d in range(DP // 16):
                    sl = pl.ds(d * 16, 16)
                    acc = rows_v[r0, sl]
                    for l in range(1, L):
                        acc = jnp.maximum(acc, rows_v[r0 + l, sl])
                    out_v[e, sl] = acc
                return carry2

            lax.fori_loop(0, CEX, ex_body, 0, unroll=False)
            pltpu.sync_copy(out_v, out_hbm.at[pl.ds(ex0, CEX)])
            return carry

        lax.fori_loop(0, chunks_per_w, chunk_body, 0, unroll=False)

    return pool_kernel, n_chunks


_TC_BM = 512


def _mlp_body(x_ref, w1_ref, b1_ref, w2_ref, b2_ref, o_ref):
    x = x_ref[...]
    h = jnp.dot(x, w1_ref[...], preferred_element_type=jnp.float32)
    h = jnp.maximum(h + b1_ref[...], 0.0)
    y = jnp.sum(h * w2_ref[...], axis=1) + b2_ref[0]
    o_ref[...] = jax.nn.sigmoid(y)


def _mlp(feats, w1p, b1, w2, b2):
    grid = (B // _TC_BM,)
    return pl.pallas_call(
        _mlp_body,
        grid=grid,
        in_specs=[
            pl.BlockSpec((_TC_BM, 2 * DP), lambda i: (i, 0)),
            pl.BlockSpec((2 * DP, HIDDEN), lambda i: (0, 0)),
            pl.BlockSpec((1, HIDDEN), lambda i: (0, 0)),
            pl.BlockSpec((1, HIDDEN), lambda i: (0, 0)),
            pl.BlockSpec(memory_space=pltpu.SMEM),
        ],
        out_specs=pl.BlockSpec((_TC_BM,), lambda i: (i,)),
        out_shape=jax.ShapeDtypeStruct((B,), jnp.float32),
    )(feats, w1p, b1.reshape(1, HIDDEN), w2.reshape(1, HIDDEN), b2)


def kernel(premise, hypothesis, emb_table, W1, b1, W2, b2):
    pool_kernel, n_chunks = _make_sc_pool()

    emb_p = jnp.pad(emb_table, ((0, 0), (0, DP - D)))
    idx = jnp.stack([premise, hypothesis], axis=1).reshape(n_chunks, ROWS_PER_CHUNK)

    feats = pool_kernel(emb_p, idx)
    feats = feats.reshape(B, 2 * DP)

    zpad = jnp.zeros((DP - D, HIDDEN), dtype=W1.dtype)
    w1p = jnp.concatenate([W1[:D], zpad, W1[D:], zpad], axis=0)

    return _mlp(feats, w1p, b1, W2, b2)


# speedup vs baseline: 6.6894x; 1.2129x over previous
import functools

import jax
import jax.numpy as jnp
from jax import lax
from jax.experimental import pallas as pl
from jax.experimental.pallas import tpu as pltpu
from jax.experimental.pallas import tpu_sc as plsc

VOCAB = 100000
D = 100
DP = 128
L = 20
B = 16384
NEX = 2 * B
HIDDEN = 4096

CEX = 16
ROWS_PER_CHUNK = CEX * L
GATHERS = ((0, 128), (128, 128), (256, 64))


def _make_sc_pool():
    info = plsc.get_sparse_core_info()
    nc, ns = info.num_cores, info.num_subcores
    nw = nc * ns
    chunks_per_w = NEX // nw // CEX
    pairs = chunks_per_w // 2

    mesh = plsc.VectorSubcoreMesh(core_axis_name="c", subcore_axis_name="s")

    @functools.partial(
        pl.kernel,
        mesh=mesh,
        out_type=jax.ShapeDtypeStruct((NEX * DP,), jnp.float32),
        scratch_types=[
            pltpu.VMEM((ROWS_PER_CHUNK,), jnp.int32),
            pltpu.VMEM((ROWS_PER_CHUNK,), jnp.int32),
            pltpu.VMEM((ROWS_PER_CHUNK, DP), jnp.float32),
            pltpu.VMEM((ROWS_PER_CHUNK, DP), jnp.float32),
            pltpu.VMEM((CEX * DP,), jnp.float32),
            pltpu.SemaphoreType.DMA,
            pltpu.SemaphoreType.DMA,
        ],
    )
    def pool_kernel(table_hbm, idx_hbm, out_hbm, idx_v0, idx_v1, rows_v0,
                    rows_v1, out_v, sem0, sem1):
        table2d = table_hbm
        wid = lax.axis_index("s") * nc + lax.axis_index("c")
        chunk0 = wid * chunks_per_w
        idx_bufs = (idx_v0, idx_v1)
        row_bufs = (rows_v0, rows_v1)
        sems = (sem0, sem1)

        def issue(buf, chunk):
            pltpu.sync_copy(
                idx_hbm.at[pl.ds(chunk * ROWS_PER_CHUNK, ROWS_PER_CHUNK)],
                idx_bufs[buf],
            )
            for off, gs in GATHERS:
                pltpu.async_copy(
                    table2d.at[idx_bufs[buf].at[pl.ds(off, gs)]],
                    row_bufs[buf].at[pl.ds(off, gs)],
                    sems[buf],
                )

        def wait_buf(buf):
            pltpu.make_async_copy(
                table2d.at[pl.ds(0, ROWS_PER_CHUNK)],
                row_bufs[buf],
                sems[buf],
            ).wait()

        def compute_store(buf, chunk):
            rows_v = row_bufs[buf]

            def ex_body(e, carry2):
                r0 = e * L
                for d in range(DP // 16):
                    sl = pl.ds(d * 16, 16)
                    acc = rows_v[r0, sl]
                    for l in range(1, L):
                        acc = jnp.maximum(acc, rows_v[r0 + l, sl])
                    out_v[pl.ds(e * DP + d * 16, 16)] = acc
                return carry2

            lax.fori_loop(0, CEX, ex_body, 0, unroll=False)
            pltpu.sync_copy(out_v, out_hbm.at[pl.ds(chunk * CEX * DP, CEX * DP)])

        issue(0, chunk0)

        def pair_body(t, carry):
            c0 = chunk0 + 2 * t
            issue(1, c0 + 1)
            wait_buf(0)
            compute_store(0, c0)

            @pl.when(t < pairs - 1)
            def _():
                issue(0, c0 + 2)

            wait_buf(1)
            compute_store(1, c0 + 1)
            return carry

        lax.fori_loop(0, pairs, pair_body, 0, unroll=False)

    return pool_kernel


_TC_BM = 512


def _mlp_body(x_ref, w1_ref, b1_ref, w2_ref, b2_ref, o_ref):
    x = x_ref[...]
    h = jnp.dot(x, w1_ref[...], preferred_element_type=jnp.float32)
    h = jnp.maximum(h + b1_ref[...], 0.0)
    y = jnp.sum(h * w2_ref[...], axis=1) + b2_ref[0]
    o_ref[...] = jax.nn.sigmoid(y)


def _mlp(feats, w1p, b1, w2, b2):
    grid = (B // _TC_BM,)
    return pl.pallas_call(
        _mlp_body,
        grid=grid,
        in_specs=[
            pl.BlockSpec((_TC_BM, 2 * DP), lambda i: (i, 0)),
            pl.BlockSpec((2 * DP, HIDDEN), lambda i: (0, 0)),
            pl.BlockSpec((1, HIDDEN), lambda i: (0, 0)),
            pl.BlockSpec((1, HIDDEN), lambda i: (0, 0)),
            pl.BlockSpec(memory_space=pltpu.SMEM),
        ],
        out_specs=pl.BlockSpec((_TC_BM,), lambda i: (i,)),
        out_shape=jax.ShapeDtypeStruct((B,), jnp.float32),
    )(feats, w1p, b1.reshape(1, HIDDEN), w2.reshape(1, HIDDEN), b2)


def kernel(premise, hypothesis, emb_table, W1, b1, W2, b2):
    pool_kernel = _make_sc_pool()

    emb_p = jnp.pad(emb_table, ((0, 0), (0, DP - D)))
    idx = jnp.stack([premise, hypothesis], axis=1).reshape(-1)

    feats = pool_kernel(emb_p, idx).reshape(B, 2 * DP)

    zpad = jnp.zeros((DP - D, HIDDEN), dtype=W1.dtype)
    w1p = jnp.concatenate([W1[:D], zpad, W1[D:], zpad], axis=0)

    return _mlp(feats, w1p, b1, W2, b2)


# speedup vs baseline: 8.8086x; 1.3168x over previous
import functools

import jax
import jax.numpy as jnp
from jax import lax
from jax.experimental import pallas as pl
from jax.experimental.pallas import tpu as pltpu
from jax.experimental.pallas import tpu_sc as plsc

VOCAB = 100000
D = 100
DP = 128
L = 20
B = 16384
NEX = 2 * B
HIDDEN = 4096

CEX = 16
ROWS_PER_CHUNK = CEX * L
GATHERS = ((0, 128), (128, 128), (256, 64))


def _make_sc_pool():
    info = plsc.get_sparse_core_info()
    nc, ns = info.num_cores, info.num_subcores
    nw = nc * ns
    chunks_per_w = NEX // nw // CEX
    pairs = chunks_per_w // 2
    idx_per_w = chunks_per_w * ROWS_PER_CHUNK

    mesh = plsc.VectorSubcoreMesh(core_axis_name="c", subcore_axis_name="s")

    @functools.partial(
        pl.kernel,
        mesh=mesh,
        out_type=jax.ShapeDtypeStruct((NEX * DP,), jnp.float32),
        scratch_types=[
            pltpu.VMEM((idx_per_w,), jnp.int32),
            pltpu.VMEM((ROWS_PER_CHUNK, DP), jnp.float32),
            pltpu.VMEM((ROWS_PER_CHUNK, DP), jnp.float32),
            pltpu.VMEM((CEX * DP,), jnp.float32),
            pltpu.SemaphoreType.DMA,
            pltpu.SemaphoreType.DMA,
        ],
    )
    def pool_kernel(table_hbm, idx_hbm, out_hbm, idx_all, rows_v0, rows_v1,
                    out_v, sem0, sem1):
        wid = lax.axis_index("s") * nc + lax.axis_index("c")
        chunk0 = wid * chunks_per_w
        row_bufs = (rows_v0, rows_v1)
        sems = (sem0, sem1)

        pltpu.sync_copy(idx_hbm.at[pl.ds(wid * idx_per_w, idx_per_w)], idx_all)

        def issue(buf, c_local):
            base = c_local * ROWS_PER_CHUNK
            for off, gs in GATHERS:
                pltpu.async_copy(
                    table_hbm.at[idx_all.at[pl.ds(base + off, gs)]],
                    row_bufs[buf].at[pl.ds(off, gs)],
                    sems[buf],
                )

        def wait_buf(buf):
            pltpu.make_async_copy(
                table_hbm.at[pl.ds(0, ROWS_PER_CHUNK)],
                row_bufs[buf],
                sems[buf],
            ).wait()

        def compute_store(buf, c_local):
            rows_v = row_bufs[buf]

            def ex_body(e, carry2):
                r0 = e * L
                for d in range(DP // 16):
                    sl = pl.ds(d * 16, 16)
                    acc = rows_v[r0, sl]
                    for l in range(1, L):
                        acc = jnp.maximum(acc, rows_v[r0 + l, sl])
                    out_v[pl.ds(e * DP + d * 16, 16)] = acc
                return carry2

            lax.fori_loop(0, CEX, ex_body, 0, unroll=False)
            pltpu.sync_copy(
                out_v,
                out_hbm.at[pl.ds((chunk0 + c_local) * CEX * DP, CEX * DP)],
            )

        issue(0, 0)

        def pair_body(t, carry):
            c0 = 2 * t
            issue(1, c0 + 1)
            wait_buf(0)
            compute_store(0, c0)

            @pl.when(t < pairs - 1)
            def _():
                issue(0, c0 + 2)

            wait_buf(1)
            compute_store(1, c0 + 1)
            return carry

        lax.fori_loop(0, pairs, pair_body, 0, unroll=False)

    return pool_kernel


_TR_BV = 2048
_TR_GRID = (VOCAB + _TR_BV - 1) // _TR_BV


def _transpose_body(xt_ref, o_ref):
    x = xt_ref[...]
    xp = jnp.concatenate(
        [x, jnp.zeros((DP - D, _TR_BV), jnp.float32)], axis=0)
    r = lax.broadcasted_iota(jnp.int32, (DP, DP), 0)
    c = lax.broadcasted_iota(jnp.int32, (DP, DP), 1)
    eye = jnp.where(r == c, 1.0, 0.0).astype(jnp.float32)
    o_ref[...] = lax.dot_general(
        xp, eye, (((0,), (0,)), ((), ())),
        preferred_element_type=jnp.float32)


def _transpose_pad(emb_t):
    return pl.pallas_call(
        _transpose_body,
        grid=(_TR_GRID,),
        in_specs=[pl.BlockSpec((D, _TR_BV), lambda i: (0, i))],
        out_specs=pl.BlockSpec((_TR_BV, DP), lambda i: (i, 0)),
        out_shape=jax.ShapeDtypeStruct((VOCAB, DP), jnp.float32),
    )(emb_t)


_TC_BM = 512


def _mlp_body(x_ref, w1_ref, b1_ref, w2_ref, b2_ref, o_ref):
    x = x_ref[...]
    h = jnp.dot(x, w1_ref[...], preferred_element_type=jnp.float32)
    h = jnp.maximum(h + b1_ref[...], 0.0)
    y = jnp.sum(h * w2_ref[...], axis=1) + b2_ref[0]
    o_ref[...] = jax.nn.sigmoid(y)


def _mlp(feats, w1p, b1, w2, b2):
    grid = (B // _TC_BM,)
    return pl.pallas_call(
        _mlp_body,
        grid=grid,
        in_specs=[
            pl.BlockSpec((_TC_BM, 2 * DP), lambda i: (i, 0)),
            pl.BlockSpec((2 * DP, HIDDEN), lambda i: (0, 0)),
            pl.BlockSpec((1, HIDDEN), lambda i: (0, 0)),
            pl.BlockSpec((1, HIDDEN), lambda i: (0, 0)),
            pl.BlockSpec(memory_space=pltpu.SMEM),
        ],
        out_specs=pl.BlockSpec((_TC_BM,), lambda i: (i,)),
        out_shape=jax.ShapeDtypeStruct((B,), jnp.float32),
    )(feats, w1p, b1.reshape(1, HIDDEN), w2.reshape(1, HIDDEN), b2)


def kernel(premise, hypothesis, emb_table, W1, b1, W2, b2):
    pool_kernel = _make_sc_pool()

    emb_p = _transpose_pad(emb_table.T)
    idx = jnp.stack([premise, hypothesis], axis=1).reshape(-1)

    feats = pool_kernel(emb_p, idx).reshape(B, 2 * DP)

    zpad = jnp.zeros((DP - D, HIDDEN), dtype=W1.dtype)
    w1p = jnp.concatenate([W1[:D], zpad, W1[D:], zpad], axis=0)

    return _mlp(feats, w1p, b1, W2, b2)


# speedup vs baseline: 8.9261x; 1.0133x over previous
import functools

import jax
import jax.numpy as jnp
from jax import lax
from jax.experimental import pallas as pl
from jax.experimental.pallas import tpu as pltpu
from jax.experimental.pallas import tpu_sc as plsc

VOCAB = 100000
D = 100
DP = 128
L = 20
B = 16384
NEX = 2 * B
HIDDEN = 4096

CEX = 16
ROWS_PER_CHUNK = CEX * L
GATHERS = ((0, 128), (128, 128), (256, 64))


def _make_sc_pool():
    info = plsc.get_sparse_core_info()
    nc, ns = info.num_cores, info.num_subcores
    nw = nc * ns
    chunks_per_w = NEX // nw // CEX
    pairs = chunks_per_w // 2
    idx_per_w = chunks_per_w * ROWS_PER_CHUNK

    mesh = plsc.VectorSubcoreMesh(core_axis_name="c", subcore_axis_name="s")

    @functools.partial(
        pl.kernel,
        mesh=mesh,
        out_type=jax.ShapeDtypeStruct((NEX * DP,), jnp.float32),
        scratch_types=[
            pltpu.VMEM((idx_per_w,), jnp.int32),
            pltpu.VMEM((ROWS_PER_CHUNK, DP), jnp.float32),
            pltpu.VMEM((ROWS_PER_CHUNK, DP), jnp.float32),
            pltpu.VMEM((CEX * DP,), jnp.float32),
            pltpu.VMEM((CEX * DP,), jnp.float32),
            pltpu.SemaphoreType.DMA,
            pltpu.SemaphoreType.DMA,
            pltpu.SemaphoreType.DMA,
            pltpu.SemaphoreType.DMA,
        ],
    )
    def pool_kernel(table_hbm, idx_hbm, out_hbm, idx_all, rows_v0, rows_v1,
                    out_v0, out_v1, sem0, sem1, osem0, osem1):
        wid = lax.axis_index("s") * nc + lax.axis_index("c")
        chunk0 = wid * chunks_per_w
        row_bufs = (rows_v0, rows_v1)
        out_bufs = (out_v0, out_v1)
        sems = (sem0, sem1)
        osems = (osem0, osem1)

        pltpu.sync_copy(idx_hbm.at[pl.ds(wid * idx_per_w, idx_per_w)], idx_all)

        def issue(buf, c_local):
            base = c_local * ROWS_PER_CHUNK
            for off, gs in GATHERS:
                pltpu.async_copy(
                    table_hbm.at[idx_all.at[pl.ds(base + off, gs)]],
                    row_bufs[buf].at[pl.ds(off, gs)],
                    sems[buf],
                )

        def wait_buf(buf):
            pltpu.make_async_copy(
                table_hbm.at[pl.ds(0, ROWS_PER_CHUNK)],
                row_bufs[buf],
                sems[buf],
            ).wait()

        def wait_out(buf):
            pltpu.make_async_copy(
                out_bufs[buf],
                out_hbm.at[pl.ds(0, CEX * DP)],
                osems[buf],
            ).wait()

        def compute_store(buf, c_local, have_outstanding):
            rows_v = row_bufs[buf]
            out_v = out_bufs[buf]

            @pl.when(have_outstanding)
            def _():
                wait_out(buf)

            def ex_body(e, carry2):
                r0 = e * L
                for d in range(DP // 16):
                    sl = pl.ds(d * 16, 16)
                    acc = rows_v[r0, sl]
                    for l in range(1, L):
                        acc = jnp.maximum(acc, rows_v[r0 + l, sl])
                    out_v[pl.ds(e * DP + d * 16, 16)] = acc
                return carry2

            lax.fori_loop(0, CEX, ex_body, 0, unroll=False)
            pltpu.async_copy(
                out_v,
                out_hbm.at[pl.ds((chunk0 + c_local) * CEX * DP, CEX * DP)],
                osems[buf],
            )

        issue(0, 0)

        def pair_body(t, carry):
            c0 = 2 * t
            issue(1, c0 + 1)
            wait_buf(0)
            compute_store(0, c0, t > 0)

            @pl.when(t < pairs - 1)
            def _():
                issue(0, c0 + 2)

            wait_buf(1)
            compute_store(1, c0 + 1, t > 0)
            return carry

        lax.fori_loop(0, pairs, pair_body, 0, unroll=False)
        wait_out(0)
        wait_out(1)

    return pool_kernel


_TR_BV = 2048
_TR_GRID = (VOCAB + _TR_BV - 1) // _TR_BV


def _transpose_body(xt_ref, o_ref):
    x = xt_ref[...]
    xp = jnp.concatenate(
        [x, jnp.zeros((DP - D, _TR_BV), jnp.float32)], axis=0)
    r = lax.broadcasted_iota(jnp.int32, (DP, DP), 0)
    c = lax.broadcasted_iota(jnp.int32, (DP, DP), 1)
    eye = jnp.where(r == c, 1.0, 0.0).astype(jnp.float32)
    o_ref[...] = lax.dot_general(
        xp, eye, (((0,), (0,)), ((), ())),
        preferred_element_type=jnp.float32)


def _transpose_pad(emb_t):
    return pl.pallas_call(
        _transpose_body,
        grid=(_TR_GRID,),
        in_specs=[pl.BlockSpec((D, _TR_BV), lambda i: (0, i))],
        out_specs=pl.BlockSpec((_TR_BV, DP), lambda i: (i, 0)),
        out_shape=jax.ShapeDtypeStruct((VOCAB, DP), jnp.float32),
    )(emb_t)


_TC_BM = 512


def _mlp_body(x_ref, w1_ref, b1_ref, w2_ref, b2_ref, o_ref):
    x = x_ref[...].astype(jnp.bfloat16)
    h = jnp.dot(x, w1_ref[...], preferred_element_type=jnp.float32)
    h = jnp.maximum(h + b1_ref[...], 0.0)
    y = jnp.sum(h * w2_ref[...], axis=1) + b2_ref[0]
    o_ref[...] = jax.nn.sigmoid(y)


def _mlp(feats, w1p, b1, w2, b2):
    grid = (B // _TC_BM,)
    return pl.pallas_call(
        _mlp_body,
        grid=grid,
        in_specs=[
            pl.BlockSpec((_TC_BM, 2 * DP), lambda i: (i, 0)),
            pl.BlockSpec((2 * DP, HIDDEN), lambda i: (0, 0)),
            pl.BlockSpec((1, HIDDEN), lambda i: (0, 0)),
            pl.BlockSpec((1, HIDDEN), lambda i: (0, 0)),
            pl.BlockSpec(memory_space=pltpu.SMEM),
        ],
        out_specs=pl.BlockSpec((_TC_BM,), lambda i: (i,)),
        out_shape=jax.ShapeDtypeStruct((B,), jnp.float32),
    )(feats, w1p, b1.reshape(1, HIDDEN), w2.reshape(1, HIDDEN), b2)


def kernel(premise, hypothesis, emb_table, W1, b1, W2, b2):
    pool_kernel = _make_sc_pool()

    emb_p = _transpose_pad(emb_table.T)
    idx = jnp.stack([premise, hypothesis], axis=1).reshape(-1)

    feats = pool_kernel(emb_p, idx).reshape(B, 2 * DP)

    zpad = jnp.zeros((DP - D, HIDDEN), dtype=W1.dtype)
    w1p = jnp.concatenate([W1[:D], zpad, W1[D:], zpad], axis=0)

    return _mlp(feats, w1p.astype(jnp.bfloat16), b1, W2, b2)


# speedup vs baseline: 9.4931x; 1.0635x over previous
import functools

import jax
import jax.numpy as jnp
from jax import lax
from jax.experimental import pallas as pl
from jax.experimental.pallas import tpu as pltpu
from jax.experimental.pallas import tpu_sc as plsc

VOCAB = 100000
D = 100
DP = 128
L = 20
B = 16384
NEX = 2 * B
HIDDEN = 4096

CEX = 16
ROWS_PER_CHUNK = CEX * L
GATHERS = ((0, 128), (128, 128), (256, 64))


NSLICE = 4


def _make_sc_pool(nex_s):
    info = plsc.get_sparse_core_info()
    nc, ns = info.num_cores, info.num_subcores
    nw = nc * ns
    chunks_per_w = nex_s // nw // CEX
    pairs = chunks_per_w // 2
    idx_per_w = chunks_per_w * ROWS_PER_CHUNK

    mesh = plsc.VectorSubcoreMesh(core_axis_name="c", subcore_axis_name="s")

    @functools.partial(
        pl.kernel,
        mesh=mesh,
        out_type=jax.ShapeDtypeStruct((nex_s * DP,), jnp.float32),
        scratch_types=[
            pltpu.VMEM((idx_per_w,), jnp.int32),
            pltpu.VMEM((ROWS_PER_CHUNK, DP), jnp.float32),
            pltpu.VMEM((ROWS_PER_CHUNK, DP), jnp.float32),
            pltpu.VMEM((CEX * DP,), jnp.float32),
            pltpu.VMEM((CEX * DP,), jnp.float32),
            pltpu.SemaphoreType.DMA,
            pltpu.SemaphoreType.DMA,
            pltpu.SemaphoreType.DMA,
            pltpu.SemaphoreType.DMA,
        ],
    )
    def pool_kernel(table_hbm, idx_hbm, out_hbm, idx_all, rows_v0, rows_v1,
                    out_v0, out_v1, sem0, sem1, osem0, osem1):
        wid = lax.axis_index("s") * nc + lax.axis_index("c")
        chunk0 = wid * chunks_per_w
        row_bufs = (rows_v0, rows_v1)
        out_bufs = (out_v0, out_v1)
        sems = (sem0, sem1)
        osems = (osem0, osem1)

        pltpu.sync_copy(idx_hbm.at[pl.ds(wid * idx_per_w, idx_per_w)], idx_all)

        def issue(buf, c_local):
            base = c_local * ROWS_PER_CHUNK
            for off, gs in GATHERS:
                pltpu.async_copy(
                    table_hbm.at[idx_all.at[pl.ds(base + off, gs)]],
                    row_bufs[buf].at[pl.ds(off, gs)],
                    sems[buf],
                )

        def wait_buf(buf):
            pltpu.make_async_copy(
                table_hbm.at[pl.ds(0, ROWS_PER_CHUNK)],
                row_bufs[buf],
                sems[buf],
            ).wait()

        def wait_out(buf):
            pltpu.make_async_copy(
                out_bufs[buf],
                out_hbm.at[pl.ds(0, CEX * DP)],
                osems[buf],
            ).wait()

        def compute_store(buf, c_local, have_outstanding):
            rows_v = row_bufs[buf]
            out_v = out_bufs[buf]

            @pl.when(have_outstanding)
            def _():
                wait_out(buf)

            def ex_body(e, carry2):
                r0 = e * L
                for d in range(DP // 16):
                    sl = pl.ds(d * 16, 16)
                    acc = rows_v[r0, sl]
                    for l in range(1, L):
                        acc = jnp.maximum(acc, rows_v[r0 + l, sl])
                    out_v[pl.ds(e * DP + d * 16, 16)] = acc
                return carry2

            lax.fori_loop(0, CEX, ex_body, 0, unroll=False)
            pltpu.async_copy(
                out_v,
                out_hbm.at[pl.ds((chunk0 + c_local) * CEX * DP, CEX * DP)],
                osems[buf],
            )

        issue(0, 0)

        def pair_body(t, carry):
            c0 = 2 * t
            issue(1, c0 + 1)
            wait_buf(0)
            compute_store(0, c0, t > 0)

            @pl.when(t < pairs - 1)
            def _():
                issue(0, c0 + 2)

            wait_buf(1)
            compute_store(1, c0 + 1, t > 0)
            return carry

        lax.fori_loop(0, pairs, pair_body, 0, unroll=False)
        wait_out(0)
        wait_out(1)

    return pool_kernel


_TR_BV = 2048
_TR_GRID = (VOCAB + _TR_BV - 1) // _TR_BV


def _transpose_body(xt_ref, o_ref):
    x = xt_ref[...]
    xp = jnp.concatenate(
        [x, jnp.zeros((DP - D, _TR_BV), jnp.float32)], axis=0)
    r = lax.broadcasted_iota(jnp.int32, (DP, DP), 0)
    c = lax.broadcasted_iota(jnp.int32, (DP, DP), 1)
    eye = jnp.where(r == c, 1.0, 0.0).astype(jnp.float32)
    o_ref[...] = lax.dot_general(
        xp, eye, (((0,), (0,)), ((), ())),
        preferred_element_type=jnp.float32)


def _transpose_pad(emb_t):
    return pl.pallas_call(
        _transpose_body,
        grid=(_TR_GRID,),
        in_specs=[pl.BlockSpec((D, _TR_BV), lambda i: (0, i))],
        out_specs=pl.BlockSpec((_TR_BV, DP), lambda i: (i, 0)),
        out_shape=jax.ShapeDtypeStruct((VOCAB, DP), jnp.float32),
    )(emb_t)


_TC_BM = 512


def _mlp_body(x_ref, w1_ref, b1_ref, w2_ref, b2_ref, o_ref):
    x = x_ref[...].astype(jnp.bfloat16)
    h = jnp.dot(x, w1_ref[...], preferred_element_type=jnp.float32)
    h = jnp.maximum(h + b1_ref[...], 0.0)
    y = jnp.sum(h * w2_ref[...], axis=1) + b2_ref[0]
    o_ref[...] = jax.nn.sigmoid(y)


def _mlp(feats, w1p, b1, w2, b2):
    bs = feats.shape[0]
    grid = (bs // _TC_BM,)
    return pl.pallas_call(
        _mlp_body,
        grid=grid,
        in_specs=[
            pl.BlockSpec((_TC_BM, 2 * DP), lambda i: (i, 0)),
            pl.BlockSpec((2 * DP, HIDDEN), lambda i: (0, 0)),
            pl.BlockSpec((1, HIDDEN), lambda i: (0, 0)),
            pl.BlockSpec((1, HIDDEN), lambda i: (0, 0)),
            pl.BlockSpec(memory_space=pltpu.SMEM),
        ],
        out_specs=pl.BlockSpec((_TC_BM,), lambda i: (i,)),
        out_shape=jax.ShapeDtypeStruct((bs,), jnp.float32),
    )(feats, w1p, b1.reshape(1, HIDDEN), w2.reshape(1, HIDDEN), b2)


def kernel(premise, hypothesis, emb_table, W1, b1, W2, b2):
    nex_s = NEX // NSLICE
    pool_kernel = _make_sc_pool(nex_s)

    emb_p = _transpose_pad(emb_table.T)
    idx = jnp.stack([premise, hypothesis], axis=1).reshape(-1)

    zpad = jnp.zeros((DP - D, HIDDEN), dtype=W1.dtype)
    w1p = jnp.concatenate([W1[:D], zpad, W1[D:], zpad], axis=0)
    w1b = w1p.astype(jnp.bfloat16)

    outs = []
    for si in range(NSLICE):
        idx_s = lax.slice(idx, (si * nex_s * L,), ((si + 1) * nex_s * L,))
        feats_s = pool_kernel(emb_p, idx_s).reshape(nex_s // 2, 2 * DP)
        outs.append(_mlp(feats_s, w1b, b1, W2, b2))
    return jnp.concatenate(outs)
